# baseline (speedup 1.0000x reference)
import jax
import jax.numpy as jnp
from jax.experimental import pallas as pl
from jax.experimental.pallas import tpu as pltpu

_ALPHA = 0.2


def _proj_kernel(x_ref, w_ref, c1_ref, c2_ref,
                 h16_ref, s1_ref, s2_ref, e2_ref, f2_ref, f1_ref):
    h = jnp.dot(x_ref[...], w_ref[...], preferred_element_type=jnp.float32)
    h16_ref[...] = h.astype(jnp.bfloat16)
    s1 = jnp.dot(h, c1_ref[...], preferred_element_type=jnp.float32)
    s2 = jnp.dot(h, c2_ref[...], preferred_element_type=jnp.float32)
    s1_ref[...] = s1
    s2_ref[...] = s2
    nz = jnp.sum(h, axis=1, keepdims=True) != 0.0
    e2_ref[...] = jnp.where(nz, jnp.exp(s2), 0.0)
    f2_ref[...] = jnp.where(nz, jnp.exp(_ALPHA * s2), 0.0)
    f1_ref[...] = jnp.exp((_ALPHA - 1.0) * s1)


def _att_kernel(adj_ref, s1_ref, s2_ref, e2_ref, f2_ref, f1_ref, rv_ref,
                h16_ref, out_ref):
    s1 = s1_ref[...]
    t = s1 + s2_ref[...]
    g = f1_ref[...] * f2_ref[...]
    w0 = jnp.where(t >= 0.0, e2_ref[...], g)
    w = jnp.where(adj_ref[...] > 0.0, w0, 0.0)
    den = jnp.sum(w, axis=1, keepdims=True)
    num = jnp.dot(w.astype(jnp.bfloat16), h16_ref[...],
                  preferred_element_type=jnp.float32)
    hp = jnp.where(den > 0.0, num / den, 0.0)
    elu = jnp.where(hp > 0.0, hp, jnp.exp(hp) - 1.0)
    out_ref[...] = elu * rv_ref[...]


def kernel(input, adj, M, W, c1, c2):
    N, Fin = input.shape
    Fout = W.shape[1]

    h16, s1, s2, e2, f2, f1 = pl.pallas_call(
        _proj_kernel,
        out_shape=[
            jax.ShapeDtypeStruct((N, Fout), jnp.bfloat16),
            jax.ShapeDtypeStruct((N, 1), jnp.float32),
            jax.ShapeDtypeStruct((N, 1), jnp.float32),
            jax.ShapeDtypeStruct((N, 1), jnp.float32),
            jax.ShapeDtypeStruct((N, 1), jnp.float32),
            jax.ShapeDtypeStruct((N, 1), jnp.float32),
        ],
    )(input, W, c1, c2)

    s2row = s2.reshape(1, N)
    e2row = e2.reshape(1, N)
    f2row = f2.reshape(1, N)
    rv = (jnp.arange(N) < M).astype(jnp.float32).reshape(N, 1)

    for cand in (200, 100, 40, 8, 1):
        if N % cand == 0:
            TM = cand
            break

    out = pl.pallas_call(
        _att_kernel,
        grid=(N // TM,),
        in_specs=[
            pl.BlockSpec((TM, N), lambda i: (i, 0)),
            pl.BlockSpec((TM, 1), lambda i: (i, 0)),
            pl.BlockSpec((1, N), lambda i: (0, 0)),
            pl.BlockSpec((1, N), lambda i: (0, 0)),
            pl.BlockSpec((1, N), lambda i: (0, 0)),
            pl.BlockSpec((TM, 1), lambda i: (i, 0)),
            pl.BlockSpec((TM, 1), lambda i: (i, 0)),
            pl.BlockSpec((N, Fout), lambda i: (0, 0)),
        ],
        out_specs=pl.BlockSpec((TM, Fout), lambda i: (i, 0)),
        out_shape=jax.ShapeDtypeStruct((N, Fout), jnp.float32),
        compiler_params=pltpu.CompilerParams(
            dimension_semantics=("parallel",),
        ),
    )(adj, s1, s2row, e2row, f2row, f1, rv, h16)
    return out


# speedup vs baseline: 1.2954x; 1.2954x over previous
import jax
import jax.numpy as jnp
from jax.experimental import pallas as pl
from jax.experimental.pallas import tpu as pltpu

_ALPHA = 0.2


def _proj_kernel(x_ref, w_ref, c1_ref, c2_ref, hext_ref, s1_ref, s2_ref, nz_ref):
    h = jnp.dot(x_ref[...], w_ref[...], preferred_element_type=jnp.float32)
    n = h.shape[0]
    hext_ref[:, :128] = h.astype(jnp.bfloat16)
    lane = jax.lax.broadcasted_iota(jnp.int32, (n, 128), 1)
    hext_ref[:, 128:] = jnp.where(lane == 0, 1.0, 0.0).astype(jnp.bfloat16)
    s1_ref[...] = jnp.dot(h, c1_ref[...], preferred_element_type=jnp.float32)
    s2_ref[...] = jnp.dot(h, c2_ref[...], preferred_element_type=jnp.float32)
    nz_ref[...] = (jnp.sum(h, axis=1, keepdims=True) != 0.0).astype(jnp.float32)


def _factor_kernel(s1_ref, s2_ref, nz_ref, e2_ref, f2_ref, q1_ref, f1_ref):
    s1 = s1_ref[...]
    s2 = s2_ref[...]
    nz = nz_ref[...]
    e2_ref[...] = nz * jnp.exp(s2)
    f2_ref[...] = nz * jnp.exp(_ALPHA * s2)
    q1_ref[...] = jnp.exp(-_ALPHA * s1)
    f1_ref[...] = jnp.exp((_ALPHA - 1.0) * s1)


def _att_kernel(adj_ref, q1_ref, f1_ref, e2_ref, f2_ref, rv_ref, hext_ref,
                out_ref):
    f2 = f2_ref[...]
    w0 = jnp.where(f2 >= q1_ref[...], e2_ref[...], f1_ref[...] * f2)
    w = jnp.where(adj_ref[...] > 0.0, w0, 0.0)
    ne = jnp.dot(w.astype(jnp.bfloat16), hext_ref[...],
                 preferred_element_type=jnp.float32)
    num = ne[:, :128]
    den = ne[:, 128:129]
    hp = jnp.where(den > 0.0, num / den, 0.0)
    elu = jnp.where(hp > 0.0, hp, jnp.exp(hp) - 1.0)
    out_ref[...] = elu * rv_ref[...]


def kernel(input, adj, M, W, c1, c2):
    N, Fin = input.shape
    Fout = W.shape[1]

    hext, s1, s2, nz = pl.pallas_call(
        _proj_kernel,
        out_shape=[
            jax.ShapeDtypeStruct((N, 2 * Fout), jnp.bfloat16),
            jax.ShapeDtypeStruct((N, 1), jnp.float32),
            jax.ShapeDtypeStruct((N, 1), jnp.float32),
            jax.ShapeDtypeStruct((N, 1), jnp.float32),
        ],
    )(input, W, c1, c2)

    e2row, f2row, q1row, f1row = pl.pallas_call(
        _factor_kernel,
        out_shape=[jax.ShapeDtypeStruct((1, N), jnp.float32)] * 4,
    )(s1.reshape(1, N), s2.reshape(1, N), nz.reshape(1, N))

    q1 = q1row.reshape(N, 1)
    f1 = f1row.reshape(N, 1)
    rv = (jnp.arange(N) < M).astype(jnp.float32).reshape(N, 1)

    for cand in (200, 100, 40, 8, 1):
        if N % cand == 0:
            TM = cand
            break

    out = pl.pallas_call(
        _att_kernel,
        grid=(N // TM,),
        in_specs=[
            pl.BlockSpec((TM, N), lambda i: (i, 0)),
            pl.BlockSpec((TM, 1), lambda i: (i, 0)),
            pl.BlockSpec((TM, 1), lambda i: (i, 0)),
            pl.BlockSpec((1, N), lambda i: (0, 0)),
            pl.BlockSpec((1, N), lambda i: (0, 0)),
            pl.BlockSpec((TM, 1), lambda i: (i, 0)),
            pl.BlockSpec((N, 2 * Fout), lambda i: (0, 0)),
        ],
        out_specs=pl.BlockSpec((TM, Fout), lambda i: (i, 0)),
        out_shape=jax.ShapeDtypeStruct((N, Fout), jnp.float32),
        compiler_params=pltpu.CompilerParams(
            dimension_semantics=("parallel",),
        ),
    )(adj, q1, f1, e2row, f2row, rv, hext)
    return out


# speedup vs baseline: 1.3347x; 1.0303x over previous
import jax
import jax.numpy as jnp
from jax.experimental import pallas as pl
from jax.experimental.pallas import tpu as pltpu

_ALPHA = 0.2


def _proj_kernel(x_ref, w_ref, c1_ref, c2_ref, hext_ref, s1_ref, s2_ref, nz_ref):
    h = jnp.dot(x_ref[...], w_ref[...], preferred_element_type=jnp.float32)
    n = h.shape[0]
    hext_ref[:, :128] = h.astype(jnp.bfloat16)
    lane = jax.lax.broadcasted_iota(jnp.int32, (n, 128), 1)
    hext_ref[:, 128:] = jnp.where(lane == 0, 1.0, 0.0).astype(jnp.bfloat16)
    s1_ref[...] = jnp.dot(h, c1_ref[...], preferred_element_type=jnp.float32)
    s2_ref[...] = jnp.dot(h, c2_ref[...], preferred_element_type=jnp.float32)
    nz_ref[...] = (jnp.sum(h, axis=1, keepdims=True) != 0.0).astype(jnp.float32)


def _factor_kernel(s1_ref, s2_ref, nz_ref, e2_ref, f2_ref, f1_ref):
    s1 = s1_ref[...]
    s2 = s2_ref[...]
    nz = nz_ref[...]
    e2_ref[...] = nz * jnp.exp(s2)
    f2_ref[...] = nz * jnp.exp(_ALPHA * s2)
    f1_ref[...] = jnp.exp((_ALPHA - 1.0) * s1)


def _att_kernel(adj_ref, f1_ref, e2_ref, f2_ref, rv_ref, hext_ref,
                out_ref):
    w0 = jnp.maximum(e2_ref[...], f1_ref[...] * f2_ref[...])
    w = jnp.where(adj_ref[...] > 0.0, w0, 0.0)
    ne = jnp.dot(w.astype(jnp.bfloat16), hext_ref[...],
                 preferred_element_type=jnp.float32)
    num = ne[:, :128]
    den = ne[:, 128:129]
    hp = jnp.where(den > 0.0, num / den, 0.0)
    elu = jnp.where(hp > 0.0, hp, jnp.exp(hp) - 1.0)
    out_ref[...] = elu * rv_ref[...]


def kernel(input, adj, M, W, c1, c2):
    N, Fin = input.shape
    Fout = W.shape[1]

    hext, s1, s2, nz = pl.pallas_call(
        _proj_kernel,
        out_shape=[
            jax.ShapeDtypeStruct((N, 2 * Fout), jnp.bfloat16),
            jax.ShapeDtypeStruct((N, 1), jnp.float32),
            jax.ShapeDtypeStruct((N, 1), jnp.float32),
            jax.ShapeDtypeStruct((N, 1), jnp.float32),
        ],
    )(input, W, c1, c2)

    e2row, f2row, f1row = pl.pallas_call(
        _factor_kernel,
        out_shape=[jax.ShapeDtypeStruct((1, N), jnp.float32)] * 3,
    )(s1.reshape(1, N), s2.reshape(1, N), nz.reshape(1, N))

    f1 = f1row.reshape(N, 1)
    rv = (jnp.arange(N) < M).astype(jnp.float32).reshape(N, 1)

    for cand in (200, 100, 40, 8, 1):
        if N % cand == 0:
            TM = cand
            break

    out = pl.pallas_call(
        _att_kernel,
        grid=(N // TM,),
        in_specs=[
            pl.BlockSpec((TM, N), lambda i: (i, 0)),
            pl.BlockSpec((TM, 1), lambda i: (i, 0)),
            pl.BlockSpec((1, N), lambda i: (0, 0)),
            pl.BlockSpec((1, N), lambda i: (0, 0)),
            pl.BlockSpec((TM, 1), lambda i: (i, 0)),
            pl.BlockSpec((N, 2 * Fout), lambda i: (0, 0)),
        ],
        out_specs=pl.BlockSpec((TM, Fout), lambda i: (i, 0)),
        out_shape=jax.ShapeDtypeStruct((N, Fout), jnp.float32),
        compiler_params=pltpu.CompilerParams(
            dimension_semantics=("parallel",),
        ),
    )(adj, f1, e2row, f2row, rv, hext)
    return out


# speedup vs baseline: 1.3423x; 1.0056x over previous
import jax
import jax.numpy as jnp
from jax.experimental import pallas as pl
from jax.experimental.pallas import tpu as pltpu

_ALPHA = 0.2


def _proj_kernel(x_ref, w_ref, c1_ref, c2_ref, hext_ref, s1_ref, s2_ref, nz_ref):
    h = jnp.dot(x_ref[...], w_ref[...], preferred_element_type=jnp.float32)
    n = h.shape[0]
    hext_ref[:, :128] = h.astype(jnp.bfloat16)
    lane = jax.lax.broadcasted_iota(jnp.int32, (n, 128), 1)
    hext_ref[:, 128:] = jnp.where(lane == 0, 1.0, 0.0).astype(jnp.bfloat16)
    s1_ref[...] = jnp.dot(h, c1_ref[...], preferred_element_type=jnp.float32)
    s2_ref[...] = jnp.dot(h, c2_ref[...], preferred_element_type=jnp.float32)
    nz_ref[...] = (jnp.sum(h, axis=1, keepdims=True) != 0.0).astype(jnp.float32)


def _factor_kernel(s1_ref, s2_ref, nz_ref, e2_ref, f2_ref, f1_ref):
    s1 = s1_ref[...]
    s2 = s2_ref[...]
    nz = nz_ref[...]
    e2_ref[...] = nz * jnp.exp(s2)
    f2_ref[...] = nz * jnp.exp(_ALPHA * s2)
    f1_ref[...] = jnp.exp((_ALPHA - 1.0) * s1)


def _att_kernel(adj_ref, f1_ref, e2_ref, f2_ref, rv_ref, hext_ref,
                out_ref):
    w0 = jnp.maximum(e2_ref[...], f1_ref[...] * f2_ref[...])
    w = jnp.where(adj_ref[...] > 0.0, w0, 0.0)
    ne = jnp.dot(w.astype(jnp.bfloat16), hext_ref[...],
                 preferred_element_type=jnp.float32)
    num = ne[:, :128]
    den = ne[:, 128:129]
    hp = jnp.where(den > 0.0, num / den, 0.0)
    elu = jnp.where(hp > 0.0, hp, jnp.exp(hp) - 1.0)
    out_ref[...] = elu * rv_ref[...]


def kernel(input, adj, M, W, c1, c2):
    N, Fin = input.shape
    Fout = W.shape[1]

    hext, s1, s2, nz = pl.pallas_call(
        _proj_kernel,
        out_shape=[
            jax.ShapeDtypeStruct((N, 2 * Fout), jnp.bfloat16),
            jax.ShapeDtypeStruct((N, 1), jnp.float32),
            jax.ShapeDtypeStruct((N, 1), jnp.float32),
            jax.ShapeDtypeStruct((N, 1), jnp.float32),
        ],
    )(input, W, c1, c2)

    e2row, f2row, f1row = pl.pallas_call(
        _factor_kernel,
        out_shape=[jax.ShapeDtypeStruct((1, N), jnp.float32)] * 3,
    )(s1.reshape(1, N), s2.reshape(1, N), nz.reshape(1, N))

    f1 = f1row.reshape(N, 1)
    rv = (jnp.arange(N) < M).astype(jnp.float32).reshape(N, 1)

    for cand in (400, 200, 100, 40, 8, 1):
        if N % cand == 0:
            TM = cand
            break

    out = pl.pallas_call(
        _att_kernel,
        grid=(N // TM,),
        in_specs=[
            pl.BlockSpec((TM, N), lambda i: (i, 0)),
            pl.BlockSpec((TM, 1), lambda i: (i, 0)),
            pl.BlockSpec((1, N), lambda i: (0, 0)),
            pl.BlockSpec((1, N), lambda i: (0, 0)),
            pl.BlockSpec((TM, 1), lambda i: (i, 0)),
            pl.BlockSpec((N, 2 * Fout), lambda i: (0, 0)),
        ],
        out_specs=pl.BlockSpec((TM, Fout), lambda i: (i, 0)),
        out_shape=jax.ShapeDtypeStruct((N, Fout), jnp.float32),
        compiler_params=pltpu.CompilerParams(
            dimension_semantics=("parallel",),
        ),
    )(adj, f1, e2row, f2row, rv, hext)
    return out


# speedup vs baseline: 1.4707x; 1.0957x over previous
import jax
import jax.numpy as jnp
from jax.experimental import pallas as pl
from jax.experimental.pallas import tpu as pltpu

_ALPHA = 0.2


def _proj_kernel(x_ref, w_ref, c1_ref, c2_ref, hext_ref, s2e_ref, f1_ref):
    h = jnp.dot(x_ref[...], w_ref[...], preferred_element_type=jnp.float32)
    n = h.shape[0]
    hext_ref[:, :128] = h.astype(jnp.bfloat16)
    lane = jax.lax.broadcasted_iota(jnp.int32, (n, 128), 1)
    hext_ref[:, 128:] = jnp.where(lane == 0, 1.0, 0.0).astype(jnp.bfloat16)
    s1 = jnp.dot(h, c1_ref[...], preferred_element_type=jnp.float32)
    s2 = jnp.dot(h, c2_ref[...], preferred_element_type=jnp.float32)
    nz = jnp.sum(h, axis=1, keepdims=True) != 0.0
    s2e_ref[...] = jnp.where(nz, s2, -1e30)
    f1_ref[...] = jnp.exp((_ALPHA - 1.0) * s1)


def _factor_kernel(s2e_ref, e2_ref, f2_ref):
    s2e = s2e_ref[...]
    e2_ref[...] = jnp.exp(s2e)
    f2_ref[...] = jnp.exp(_ALPHA * s2e)


def _att_kernel(adj_ref, f1_ref, e2_ref, f2_ref, rv_ref, hext_ref,
                out_ref):
    w0 = jnp.maximum(e2_ref[...], f1_ref[...] * f2_ref[...])
    w = jnp.where(adj_ref[...] > 0.0, w0, 0.0)
    ne = jnp.dot(w.astype(jnp.bfloat16), hext_ref[...],
                 preferred_element_type=jnp.float32)
    num = ne[:, :128]
    den = ne[:, 128:129]
    hp = jnp.where(den > 0.0, num / den, 0.0)
    elu = jnp.where(hp > 0.0, hp, jnp.exp(hp) - 1.0)
    out_ref[...] = elu * rv_ref[...]


def kernel(input, adj, M, W, c1, c2):
    N, Fin = input.shape
    Fout = W.shape[1]

    hext, s2e, f1 = pl.pallas_call(
        _proj_kernel,
        out_shape=[
            jax.ShapeDtypeStruct((N, 2 * Fout), jnp.bfloat16),
            jax.ShapeDtypeStruct((N, 1), jnp.float32),
            jax.ShapeDtypeStruct((N, 1), jnp.float32),
        ],
    )(input, W, c1, c2)

    e2row, f2row = pl.pallas_call(
        _factor_kernel,
        out_shape=[jax.ShapeDtypeStruct((1, N), jnp.float32)] * 2,
    )(s2e.reshape(1, N))
    rv = (jnp.arange(N) < M).astype(jnp.float32).reshape(N, 1)

    for cand in (400, 200, 100, 40, 8, 1):
        if N % cand == 0:
            TM = cand
            break

    out = pl.pallas_call(
        _att_kernel,
        grid=(N // TM,),
        in_specs=[
            pl.BlockSpec((TM, N), lambda i: (i, 0)),
            pl.BlockSpec((TM, 1), lambda i: (i, 0)),
            pl.BlockSpec((1, N), lambda i: (0, 0)),
            pl.BlockSpec((1, N), lambda i: (0, 0)),
            pl.BlockSpec((TM, 1), lambda i: (i, 0)),
            pl.BlockSpec((N, 2 * Fout), lambda i: (0, 0)),
        ],
        out_specs=pl.BlockSpec((TM, Fout), lambda i: (i, 0)),
        out_shape=jax.ShapeDtypeStruct((N, Fout), jnp.float32),
        compiler_params=pltpu.CompilerParams(
            dimension_semantics=("parallel",),
        ),
    )(adj, f1, e2row, f2row, rv, hext)
    return out
